# baseline (speedup 1.0000x reference)
import functools
import math

import jax
import jax.numpy as jnp
from jax import lax
from jax.experimental import pallas as pl
from jax.experimental.pallas import tpu as pltpu
from jax.experimental.pallas import tpu_sc as plsc

_B = 16
_MAX_NODE = 2048
_C = 256
_N_TOK = 16384
_NPF = 128
_TEMPERATURE = 10000.0

_NC = 2
_NS = 16
_ROWS_PER_W = (_B * _MAX_NODE) // (_NC * _NS)
_CHUNKS = (1024, 512, 256, 128, 64, 32, 16, 8, 4, 2, 1)


def _sc_body(feat_hbm, cxy_hbm, zf_hbm, zc_hbm, cu_lo_hbm, cu_hi_hbm,
             out_hbm, pxy_hbm, cu_lo_v, cu_hi_v):
    b = lax.axis_index("s")
    half = lax.axis_index("c")
    pltpu.sync_copy(cu_lo_hbm, cu_lo_v)
    pltpu.sync_copy(cu_hi_hbm, cu_hi_v)
    onehot = lax.broadcasted_iota(jnp.int32, (16,), 0) == b
    neg = jnp.int32(-(2 ** 31))
    cu_b = jnp.max(jnp.where(onehot, cu_lo_v[...], neg))
    cu_b1 = jnp.max(jnp.where(onehot, cu_hi_v[...], neg))
    keep = jnp.minimum(cu_b1 - cu_b, _MAX_NODE)
    r0 = half * _ROWS_PER_W
    ncopy = jnp.clip(keep - r0, 0, _ROWS_PER_W)
    src0 = cu_b + r0
    dst0 = b * _MAX_NODE + r0

    done = jnp.int32(0)
    for sz in _CHUNKS:
        hit = (ncopy & sz) != 0

        @pl.when(hit)
        def _(sz=sz, done=done):
            pltpu.sync_copy(feat_hbm.at[pl.ds(src0 + done, sz), :],
                            out_hbm.at[pl.ds(dst0 + done, sz), :])
            pltpu.sync_copy(cxy_hbm.at[pl.ds(src0 + done, sz), :],
                            pxy_hbm.at[pl.ds(dst0 + done, sz), :])

        done = done + jnp.where(hit, sz, 0).astype(jnp.int32)

    nzero = _ROWS_PER_W - ncopy
    zstart = dst0 + ncopy
    zdone = jnp.int32(0)
    for sz in _CHUNKS:
        hit = (nzero & sz) != 0

        @pl.when(hit)
        def _(sz=sz, zdone=zdone):
            pltpu.sync_copy(zf_hbm.at[pl.ds(0, sz), :],
                            out_hbm.at[pl.ds(zstart + zdone, sz), :])
            pltpu.sync_copy(zc_hbm.at[pl.ds(0, sz), :],
                            pxy_hbm.at[pl.ds(zstart + zdone, sz), :])

        zdone = zdone + jnp.where(hit, sz, 0).astype(jnp.int32)


@functools.cache
def _sc_pad_copy():
    return pl.kernel(
        _sc_body,
        out_type=(
            jax.ShapeDtypeStruct((_B * _MAX_NODE, _C), jnp.float32),
            jax.ShapeDtypeStruct((_B * _MAX_NODE, 2), jnp.float32),
        ),
        mesh=plsc.VectorSubcoreMesh(core_axis_name="c", subcore_axis_name="s",
                                    num_cores=_NC, num_subcores=_NS),
        scratch_types=[
            pltpu.VMEM((16,), jnp.int32),
            pltpu.VMEM((16,), jnp.int32),
        ],
        compiler_params=pltpu.CompilerParams(use_tc_tiling_on_sc=False,
                                             needs_layout_passes=False),
    )


def _tc_body(cu_smem, pxy_ref, emd_ref, mask_ref):
    b = pl.program_id(0)
    cu_b = cu_smem[b]
    n_b = cu_smem[b + 1] - cu_b
    keep = jnp.minimum(n_b, _MAX_NODE)

    xy = pxy_ref[0]
    xmax = jnp.max(xy[:, 0:1])
    ymax = jnp.max(xy[:, 1:2])
    scale = 2.0 * math.pi
    kx = scale / (xmax + 1e-6)
    ky = scale / (ymax + 1e-6)

    j = lax.broadcasted_iota(jnp.int32, (1, _NPF // 2), 1).astype(jnp.float32)
    inv_d = jnp.exp(j * (-math.log(_TEMPERATURE) / (_NPF // 2)))

    ch = 256

    def chunk(i, carry):
        sub = pxy_ref[0, pl.ds(i * ch, ch), :]
        ax = (sub[:, 0:1] * kx) * inv_d
        ay = (sub[:, 1:2] * ky) * inv_d
        px = jnp.stack([jnp.sin(ax), jnp.cos(ax)], axis=-1).reshape(ch, _NPF)
        py = jnp.stack([jnp.sin(ay), jnp.cos(ay)], axis=-1).reshape(ch, _NPF)
        emd_ref[0, pl.ds(i * ch, ch), :] = jnp.concatenate([py, px], axis=-1)
        return carry

    lax.fori_loop(0, _MAX_NODE // ch, chunk, 0)

    col = lax.broadcasted_iota(jnp.int32, (1, 1, _MAX_NODE), 2)
    mask_ref[...] = jnp.where((n_b <= _MAX_NODE) & (col >= n_b + 1), 1.0, 0.0)


def _tc_emd(cu, pxy):
    return pl.pallas_call(
        _tc_body,
        grid=(_B,),
        in_specs=[
            pl.BlockSpec(memory_space=pltpu.SMEM),
            pl.BlockSpec((1, _MAX_NODE, 2), lambda b: (b, 0, 0)),
        ],
        out_specs=[
            pl.BlockSpec((1, _MAX_NODE, _C), lambda b: (b, 0, 0)),
            pl.BlockSpec((1, 1, _MAX_NODE), lambda b: (b, 0, 0)),
        ],
        out_shape=[
            jax.ShapeDtypeStruct((_B, _MAX_NODE, _C), jnp.float32),
            jax.ShapeDtypeStruct((_B, 1, _MAX_NODE), jnp.float32),
        ],
    )(cu, pxy)


def kernel(features, centroids, cu_seqlens):
    cu = cu_seqlens.astype(jnp.int32)
    zf = jnp.zeros((_ROWS_PER_W, _C), jnp.float32)
    zc = jnp.zeros((_ROWS_PER_W, 2), jnp.float32)
    feats, pxy = _sc_pad_copy()(features, centroids, zf, zc, cu[:_B], cu[1:_B + 1])
    pos_emd, mask = _tc_emd(cu, pxy.reshape(_B, _MAX_NODE, 2))
    return feats.reshape(_B, _MAX_NODE, _C), mask.reshape(_B, _MAX_NODE), pos_emd


# speedup vs baseline: 1.5103x; 1.5103x over previous
import functools
import math

import jax
import jax.numpy as jnp
from jax import lax
from jax.experimental import pallas as pl
from jax.experimental.pallas import tpu as pltpu
from jax.experimental.pallas import tpu_sc as plsc

_B = 16
_MAX_NODE = 2048
_C = 256
_N_TOK = 16384
_NPF = 128
_TEMPERATURE = 10000.0

_NC = 2
_NS = 16
_ROWS_PER_W = (_B * _MAX_NODE) // (_NC * _NS)
_CHUNKS = (1024, 512, 256, 128, 64, 32, 16, 8, 4, 2, 1)


_CHD = 256
_CHZ = 128


def _sc_body(feat_hbm, cxy_hbm, zf_hbm, zc_hbm, cu_lo_hbm, cu_hi_hbm,
             out_hbm, pxy_hbm, cu_lo_v, cu_hi_v, dbuf, cbuf, zbuf, czbuf):
    b = lax.axis_index("s")
    half = lax.axis_index("c")
    pltpu.sync_copy(cu_lo_hbm, cu_lo_v)
    pltpu.sync_copy(cu_hi_hbm, cu_hi_v)
    onehot = lax.broadcasted_iota(jnp.int32, (16,), 0) == b
    neg = jnp.int32(-(2 ** 31))
    cu_b = jnp.max(jnp.where(onehot, cu_lo_v[...], neg))
    cu_b1 = jnp.max(jnp.where(onehot, cu_hi_v[...], neg))
    keep = jnp.minimum(cu_b1 - cu_b, _MAX_NODE)
    r0 = half * _ROWS_PER_W
    ncopy = jnp.clip(keep - r0, 0, _ROWS_PER_W)
    src0 = cu_b + r0
    dst0 = b * _MAX_NODE + r0

    pltpu.sync_copy(zf_hbm, zbuf)
    pltpu.sync_copy(zc_hbm, czbuf)

    def pair(src, dst, sz):
        pltpu.sync_copy(feat_hbm.at[pl.ds(src, sz), :], dbuf.at[pl.ds(0, sz), :])
        pltpu.sync_copy(dbuf.at[pl.ds(0, sz), :], out_hbm.at[pl.ds(dst, sz), :])
        pltpu.sync_copy(cxy_hbm.at[pl.ds(src, sz), :], cbuf.at[pl.ds(0, sz), :])
        pltpu.sync_copy(cbuf.at[pl.ds(0, sz), :], pxy_hbm.at[pl.ds(dst, sz), :])

    nfull = ncopy // _CHD

    def data_chunk(q, carry):
        pair(src0 + q * _CHD, dst0 + q * _CHD, _CHD)
        return carry

    lax.fori_loop(0, nfull, data_chunk, 0)

    ragged = (ncopy % _CHD) != 0

    @pl.when(ragged & (ncopy >= _CHD))
    def _():
        pair(src0 + ncopy - _CHD, dst0 + ncopy - _CHD, _CHD)

    @pl.when(ragged & (ncopy < _CHD))
    def _():
        done = jnp.int32(0)
        for sz in (128, 64, 32, 16, 8, 4, 2, 1):
            hit = (ncopy & sz) != 0

            @pl.when(hit)
            def _(sz=sz, done=done):
                pair(src0 + done, dst0 + done, sz)

            done = done + jnp.where(hit, sz, 0).astype(jnp.int32)

    nzero = _ROWS_PER_W - ncopy
    zstart = dst0 + ncopy
    zb = (_CHZ - ncopy % _CHZ) % _CHZ
    zhead = jnp.minimum(zb, nzero)
    zoff = jnp.int32(0)
    for sz in (64, 32, 16, 8, 4, 2, 1):
        hit = (zhead & sz) != 0

        @pl.when(hit)
        def _(sz=sz, zoff=zoff):
            pltpu.sync_copy(zbuf.at[pl.ds(0, sz), :],
                            out_hbm.at[pl.ds(zstart + zoff, sz), :])
            pltpu.sync_copy(czbuf.at[pl.ds(0, sz), :],
                            pxy_hbm.at[pl.ds(zstart + zoff, sz), :])

        zoff = zoff + jnp.where(hit, sz, 0).astype(jnp.int32)

    def zero_chunk(q, carry):
        d = zstart + zhead + q * _CHZ
        pltpu.sync_copy(zbuf, out_hbm.at[pl.ds(d, _CHZ), :])
        pltpu.sync_copy(czbuf, pxy_hbm.at[pl.ds(d, _CHZ), :])
        return carry

    lax.fori_loop(0, (nzero - zhead) // _CHZ, zero_chunk, 0)


@functools.cache
def _sc_pad_copy():
    return pl.kernel(
        _sc_body,
        out_type=(
            jax.ShapeDtypeStruct((_B * _MAX_NODE, _C), jnp.float32),
            jax.ShapeDtypeStruct((_B * _MAX_NODE, 2), jnp.float32),
        ),
        mesh=plsc.VectorSubcoreMesh(core_axis_name="c", subcore_axis_name="s",
                                    num_cores=_NC, num_subcores=_NS),
        scratch_types=[
            pltpu.VMEM((16,), jnp.int32),
            pltpu.VMEM((16,), jnp.int32),
            pltpu.VMEM((_CHD, _C), jnp.float32),
            pltpu.VMEM((_CHD, 2), jnp.float32),
            pltpu.VMEM((_CHZ, _C), jnp.float32),
            pltpu.VMEM((_CHZ, 2), jnp.float32),
        ],
        compiler_params=pltpu.CompilerParams(use_tc_tiling_on_sc=False,
                                             needs_layout_passes=False),
    )


def _tc_body(cu_smem, pxy_ref, emd_ref, mask_ref):
    b = pl.program_id(0)
    cu_b = cu_smem[b]
    n_b = cu_smem[b + 1] - cu_b
    keep = jnp.minimum(n_b, _MAX_NODE)

    xy = pxy_ref[0]
    xmax = jnp.max(xy[:, 0:1])
    ymax = jnp.max(xy[:, 1:2])
    scale = 2.0 * math.pi
    kx = scale / (xmax + 1e-6)
    ky = scale / (ymax + 1e-6)

    j = lax.broadcasted_iota(jnp.int32, (1, _NPF // 2), 1).astype(jnp.float32)
    inv_d = jnp.exp(j * (-math.log(_TEMPERATURE) / (_NPF // 2)))

    ch = 256

    def chunk(i, carry):
        sub = pxy_ref[0, pl.ds(i * ch, ch), :]
        ax = (sub[:, 0:1] * kx) * inv_d
        ay = (sub[:, 1:2] * ky) * inv_d
        px = jnp.stack([jnp.sin(ax), jnp.cos(ax)], axis=-1).reshape(ch, _NPF)
        py = jnp.stack([jnp.sin(ay), jnp.cos(ay)], axis=-1).reshape(ch, _NPF)
        emd_ref[0, pl.ds(i * ch, ch), :] = jnp.concatenate([py, px], axis=-1)
        return carry

    lax.fori_loop(0, _MAX_NODE // ch, chunk, 0)

    col = lax.broadcasted_iota(jnp.int32, (1, 1, _MAX_NODE), 2)
    mask_ref[...] = jnp.where((n_b <= _MAX_NODE) & (col >= n_b + 1), 1.0, 0.0)


def _tc_emd(cu, pxy):
    return pl.pallas_call(
        _tc_body,
        grid=(_B,),
        in_specs=[
            pl.BlockSpec(memory_space=pltpu.SMEM),
            pl.BlockSpec((1, _MAX_NODE, 2), lambda b: (b, 0, 0)),
        ],
        out_specs=[
            pl.BlockSpec((1, _MAX_NODE, _C), lambda b: (b, 0, 0)),
            pl.BlockSpec((1, 1, _MAX_NODE), lambda b: (b, 0, 0)),
        ],
        out_shape=[
            jax.ShapeDtypeStruct((_B, _MAX_NODE, _C), jnp.float32),
            jax.ShapeDtypeStruct((_B, 1, _MAX_NODE), jnp.float32),
        ],
    )(cu, pxy)


def kernel(features, centroids, cu_seqlens):
    cu = cu_seqlens.astype(jnp.int32)
    zf = jnp.zeros((_CHZ, _C), jnp.float32)
    zc = jnp.zeros((_CHZ, 2), jnp.float32)
    feats, pxy = _sc_pad_copy()(features, centroids, zf, zc, cu[:_B], cu[1:_B + 1])
    pos_emd, mask = _tc_emd(cu, pxy.reshape(_B, _MAX_NODE, 2))
    return feats.reshape(_B, _MAX_NODE, _C), mask.reshape(_B, _MAX_NODE), pos_emd


# speedup vs baseline: 20.4628x; 13.5492x over previous
import functools
import math

import jax
import jax.numpy as jnp
from jax import lax
from jax.experimental import pallas as pl
from jax.experimental.pallas import tpu as pltpu
from jax.experimental.pallas import tpu_sc as plsc

_B = 16
_MAX_NODE = 2048
_C = 256
_N_TOK = 16384
_NPF = 128
_TEMPERATURE = 10000.0

_NC = 2
_NS = 16
_ROWS_PER_W = (_B * _MAX_NODE) // (_NC * _NS)
_CHD = 256
_CHZ = 128


def _worker_extent(cu_lo_hbm, cu_hi_hbm, cu_lo_v, cu_hi_v):
    b = lax.axis_index("s")
    half = lax.axis_index("c")
    pltpu.sync_copy(cu_lo_hbm, cu_lo_v)
    pltpu.sync_copy(cu_hi_hbm, cu_hi_v)
    onehot = lax.broadcasted_iota(jnp.int32, (16,), 0) == b
    neg = jnp.int32(-(2 ** 31))
    cu_b = jnp.max(jnp.where(onehot, cu_lo_v[...], neg))
    cu_b1 = jnp.max(jnp.where(onehot, cu_hi_v[...], neg))
    keep = jnp.minimum(cu_b1 - cu_b, _MAX_NODE)
    r0 = half * _ROWS_PER_W
    ncopy = jnp.clip(keep - r0, 0, _ROWS_PER_W)
    return cu_b + r0, b * _MAX_NODE + r0, ncopy


def _routed_copy(src_hbm, dst_hbm, buf, zbuf, src0, dst0, ncopy, chd, chz):
    def pair(src, dst, sz):
        pltpu.sync_copy(src_hbm.at[pl.ds(src, sz), :], buf.at[pl.ds(0, sz), :])
        pltpu.sync_copy(buf.at[pl.ds(0, sz), :], dst_hbm.at[pl.ds(dst, sz), :])

    def data_chunk(q, carry):
        pair(src0 + q * chd, dst0 + q * chd, chd)
        return carry

    lax.fori_loop(0, ncopy // chd, data_chunk, 0)

    ragged = (ncopy % chd) != 0

    @pl.when(ragged & (ncopy >= chd))
    def _():
        pair(src0 + ncopy - chd, dst0 + ncopy - chd, chd)

    @pl.when(ragged & (ncopy < chd))
    def _():
        done = jnp.int32(0)
        for sz in (chd // 2, chd // 4, chd // 8):
            if sz < 1:
                continue
            hit = (ncopy & sz) != 0

            @pl.when(hit)
            def _(sz=sz, done=done):
                pair(src0 + done, dst0 + done, sz)

            done = done + jnp.where(hit, sz, 0).astype(jnp.int32)
        for sz in [s for s in (64, 32, 16, 8, 4, 2, 1) if s < chd // 8]:
            hit = (ncopy & sz) != 0

            @pl.when(hit)
            def _(sz=sz, done=done):
                pair(src0 + done, dst0 + done, sz)

            done = done + jnp.where(hit, sz, 0).astype(jnp.int32)

    nzero = _ROWS_PER_W - ncopy
    zstart = dst0 + ncopy
    zb = (chz - ncopy % chz) % chz
    zhead = jnp.minimum(zb, nzero)
    zoff = jnp.int32(0)
    for sz in [s for s in (512, 256, 128, 64, 32, 16, 8, 4, 2, 1) if s < chz]:
        hit = (zhead & sz) != 0

        @pl.when(hit)
        def _(sz=sz, zoff=zoff):
            pltpu.sync_copy(zbuf.at[pl.ds(0, sz), :],
                            dst_hbm.at[pl.ds(zstart + zoff, sz), :])

        zoff = zoff + jnp.where(hit, sz, 0).astype(jnp.int32)

    def zero_chunk(q, carry):
        pltpu.sync_copy(zbuf, dst_hbm.at[pl.ds(zstart + zhead + q * chz, chz), :])
        return carry

    lax.fori_loop(0, (nzero - zhead) // chz, zero_chunk, 0)


def _sc_feats_body(feat_hbm, zf_hbm, cu_lo_hbm, cu_hi_hbm,
                   out_hbm, cu_lo_v, cu_hi_v, dbuf, zbuf):
    src0, dst0, ncopy = _worker_extent(cu_lo_hbm, cu_hi_hbm, cu_lo_v, cu_hi_v)
    pltpu.sync_copy(zf_hbm, zbuf)
    _routed_copy(feat_hbm, out_hbm, dbuf, zbuf, src0, dst0, ncopy, _CHD, _CHZ)


def _sc_pxy_body(cxy_hbm, zc_hbm, cu_lo_hbm, cu_hi_hbm,
                 pxy_hbm, cu_lo_v, cu_hi_v, cbuf, sem):
    src0, dst0, ncopy = _worker_extent(cu_lo_hbm, cu_hi_hbm, cu_lo_v, cu_hi_v)
    done = jnp.int32(0)
    for sz in (1024, 512, 256, 128, 64, 32, 16, 8, 4, 2, 1):
        hit = (ncopy & sz) != 0

        @pl.when(hit)
        def _(sz=sz, done=done):
            pltpu.async_copy(cxy_hbm.at[pl.ds(src0 + done, sz), :],
                             cbuf.at[pl.ds(done, sz), :], sem)

        done = done + jnp.where(hit, sz, 0).astype(jnp.int32)
    nzero = _ROWS_PER_W - ncopy
    zoff = jnp.int32(0)
    for sz in (1024, 512, 256, 128, 64, 32, 16, 8, 4, 2, 1):
        hit = (nzero & sz) != 0

        @pl.when(hit)
        def _(sz=sz, zoff=zoff):
            pltpu.async_copy(zc_hbm.at[pl.ds(0, sz), :],
                             cbuf.at[pl.ds(ncopy + zoff, sz), :], sem)

        zoff = zoff + jnp.where(hit, sz, 0).astype(jnp.int32)
    pltpu.make_async_copy(zc_hbm, cbuf, sem).wait()
    pltpu.sync_copy(cbuf, pxy_hbm.at[pl.ds(dst0, _ROWS_PER_W), :])


def _sc_mesh():
    return plsc.VectorSubcoreMesh(core_axis_name="c", subcore_axis_name="s",
                                  num_cores=_NC, num_subcores=_NS)


_SC_PARAMS = dict(
    compiler_params=pltpu.CompilerParams(use_tc_tiling_on_sc=False,
                                         needs_layout_passes=False),
)


@functools.cache
def _sc_feats_copy():
    return pl.kernel(
        _sc_feats_body,
        out_type=jax.ShapeDtypeStruct((_B * _MAX_NODE, _C), jnp.float32),
        mesh=_sc_mesh(),
        scratch_types=[
            pltpu.VMEM((16,), jnp.int32),
            pltpu.VMEM((16,), jnp.int32),
            pltpu.VMEM((_CHD, _C), jnp.float32),
            pltpu.VMEM((_CHZ, _C), jnp.float32),
        ],
        **_SC_PARAMS,
    )


@functools.cache
def _sc_pxy_copy():
    return pl.kernel(
        _sc_pxy_body,
        out_type=jax.ShapeDtypeStruct((_B * _MAX_NODE, 2), jnp.float32),
        mesh=_sc_mesh(),
        scratch_types=[
            pltpu.VMEM((16,), jnp.int32),
            pltpu.VMEM((16,), jnp.int32),
            pltpu.VMEM((_ROWS_PER_W, 2), jnp.float32),
            pltpu.SemaphoreType.DMA,
        ],
        **_SC_PARAMS,
    )


def _tc_body(cu_smem, pxy_ref, emd_ref, mask_ref):
    b = pl.program_id(0)
    n_b = cu_smem[b + 1] - cu_smem[b]

    xy = pxy_ref[0]
    xmax = jnp.max(xy[:, 0:1])
    ymax = jnp.max(xy[:, 1:2])
    scale = 2.0 * math.pi
    kx = scale / (xmax + 1e-6)
    ky = scale / (ymax + 1e-6)

    k = lax.broadcasted_iota(jnp.int32, (1, _NPF), 1)
    jh = (k >> 1).astype(jnp.float32)
    inv_dim = jnp.exp(jh * (-2.0 * math.log(_TEMPERATURE) / _NPF))
    off = (k & 1).astype(jnp.float32) * (0.5 * math.pi)

    pi = math.pi

    def fast_sin(u):
        t = u - pi
        t = t - jnp.where(t > pi, 2.0 * pi, 0.0)
        t2 = t * t
        p = jnp.float32(2.173256960049e-06)
        p = p * t2 + jnp.float32(-1.931626988860e-04)
        p = p * t2 + jnp.float32(8.312388279693e-03)
        p = p * t2 + jnp.float32(-1.666325937682e-01)
        p = p * t2 + jnp.float32(9.999845934511e-01)
        return -(p * t)

    ch = 512
    for i in range(_MAX_NODE // ch):
        sub = xy[i * ch:(i + 1) * ch]
        x_e = sub[:, 0:1] * kx
        y_e = sub[:, 1:2] * ky
        emd_ref[0, pl.ds(i * ch, ch), 0:_NPF] = fast_sin(y_e * inv_dim + off)
        emd_ref[0, pl.ds(i * ch, ch), _NPF:_C] = fast_sin(x_e * inv_dim + off)

    col = lax.broadcasted_iota(jnp.int32, (1, 1, _MAX_NODE), 2)
    mask_ref[...] = jnp.where((n_b <= _MAX_NODE) & (col >= n_b + 1), 1.0, 0.0)


def _tc_emd(cu, pxy):
    return pl.pallas_call(
        _tc_body,
        grid=(_B,),
        in_specs=[
            pl.BlockSpec(memory_space=pltpu.SMEM),
            pl.BlockSpec((1, _MAX_NODE, 2), lambda b: (b, 0, 0)),
        ],
        out_specs=[
            pl.BlockSpec((1, _MAX_NODE, _C), lambda b: (b, 0, 0)),
            pl.BlockSpec((1, 1, _MAX_NODE), lambda b: (b, 0, 0)),
        ],
        out_shape=[
            jax.ShapeDtypeStruct((_B, _MAX_NODE, _C), jnp.float32),
            jax.ShapeDtypeStruct((_B, 1, _MAX_NODE), jnp.float32),
        ],
    )(cu, pxy)


def kernel(features, centroids, cu_seqlens):
    cu = cu_seqlens.astype(jnp.int32)
    cu_lo, cu_hi = cu[:_B], cu[1:_B + 1]
    zf = jnp.zeros((_CHZ, _C), jnp.float32)
    zc = jnp.zeros((_ROWS_PER_W, 2), jnp.float32)
    pxy = _sc_pxy_copy()(centroids, zc, cu_lo, cu_hi)
    feats = _sc_feats_copy()(features, zf, cu_lo, cu_hi)
    pos_emd, mask = _tc_emd(cu, pxy.reshape(_B, _MAX_NODE, 2))
    return feats.reshape(_B, _MAX_NODE, _C), mask.reshape(_B, _MAX_NODE), pos_emd
